# baseline (device time: 94378 ns/iter reference)
import functools

import jax
import jax.numpy as jnp
from jax import lax
from jax.experimental import pallas as pl
from jax.experimental.pallas import tpu as pltpu

N_DEV = 4
N_TOK = 2048
D_MODEL = 1024
N_EXP = 32
EXP_PER_DEV = N_EXP // N_DEV
CAPACITY = 51
CAP_PAD = 64


def _ring_allgather(y_shard):
    m_per, n = y_shard.shape

    def body(y_ref, out_ref, comm_ref, send_sems, recv_sems):
        my_pos = lax.axis_index("i")
        left = (my_pos - 1) % N_DEV
        right = (my_pos + 1) % N_DEV

        barrier_sem = pltpu.get_barrier_semaphore()
        for nbr in [left, right]:
            pl.semaphore_signal(
                barrier_sem, inc=1,
                device_id=(nbr,), device_id_type=pl.DeviceIdType.MESH,
            )
        pl.semaphore_wait(barrier_sem, 2)

        out_ref[pl.ds(my_pos * m_per, m_per), :] = y_ref[:, :]
        comm_ref[0, :, :] = y_ref[:, :]

        for h in range(N_DEV - 1):
            send_slot = h % 2
            recv_slot = (h + 1) % 2
            rdma = pltpu.make_async_remote_copy(
                src_ref=comm_ref.at[send_slot],
                dst_ref=comm_ref.at[recv_slot],
                send_sem=send_sems.at[send_slot],
                recv_sem=recv_sems.at[recv_slot],
                device_id=(right,),
                device_id_type=pl.DeviceIdType.MESH,
            )
            rdma.start()
            rdma.wait()

            origin = (my_pos - h - 1) % N_DEV
            out_ref[pl.ds(origin * m_per, m_per), :] = comm_ref[recv_slot, :, :]

    return pl.pallas_call(
        body,
        out_shape=jax.ShapeDtypeStruct((N_DEV * m_per, n), y_shard.dtype),
        in_specs=[pl.BlockSpec(memory_space=pltpu.VMEM)],
        out_specs=pl.BlockSpec(memory_space=pltpu.VMEM),
        scratch_shapes=[
            pltpu.VMEM((2, m_per, n), y_shard.dtype),
            pltpu.SemaphoreType.DMA((2,)),
            pltpu.SemaphoreType.DMA((2,)),
        ],
        compiler_params=pltpu.CompilerParams(collective_id=0),
    )(y_shard)


def kernel(x, router_W, route_idx, expert_W):
    del router_W
    my_i = lax.axis_index("i")

    e = route_idx[:, 0]
    order = jnp.argsort(e, stable=True)
    counts = jnp.bincount(e, length=N_EXP)
    starts = jnp.cumsum(counts) - counts
    kept = jnp.minimum(counts, CAPACITY)

    slot = jnp.arange(CAP_PAD)[None, :]
    pos = starts[:, None] + slot
    valid = slot < kept[:, None]
    tok = jnp.take(order, jnp.minimum(pos, N_TOK - 1), axis=0)
    tok = jnp.where(valid, tok, 0)

    tok_loc = lax.dynamic_slice_in_dim(tok, my_i * EXP_PER_DEV, EXP_PER_DEV, 0)
    xg = jnp.take(x, tok_loc.reshape(-1), axis=0).astype(jnp.bfloat16)
    y = jnp.einsum(
        "ecd,edf->ecf",
        xg.reshape(EXP_PER_DEV, CAP_PAD, D_MODEL),
        expert_W.astype(jnp.bfloat16),
        preferred_element_type=jnp.bfloat16,
    )
    y_local = y.reshape(EXP_PER_DEV * CAP_PAD, D_MODEL)

    y_all = _ring_allgather(y_local)

    tok_f = tok.reshape(-1)
    valid_f = valid.reshape(-1)
    contrib = jnp.where(valid_f[:, None], y_all.astype(jnp.float32), 0.0)
    out = jnp.zeros((N_TOK, D_MODEL), jnp.float32).at[tok_f].add(contrib)
    return out


# device time: 82208 ns/iter; 1.1480x vs baseline; 1.1480x over previous
import jax
import jax.numpy as jnp
from jax import lax
from jax.experimental import pallas as pl
from jax.experimental.pallas import tpu as pltpu

N_DEV = 4
N_TOK = 2048
D_MODEL = 1024
N_EXP = 32
EXP_PER_DEV = N_EXP // N_DEV
CAPACITY = 51
CAP_PAD = 64
ROWS = EXP_PER_DEV * CAP_PAD
HALF = ROWS // 2


def _moe_body(x_ref, disp_ref, unperm_ref, w_hbm_ref, out_ref,
              y_all, xg, w_buf, w_sems, send_sems, recv_sems):
    my_pos = lax.axis_index("i")
    left = (my_pos - 1) % N_DEV
    right = (my_pos + 1) % N_DEV
    diag = (my_pos + 2) % N_DEV

    barrier_sem = pltpu.get_barrier_semaphore()
    for nbr in [left, right]:
        pl.semaphore_signal(
            barrier_sem, inc=1,
            device_id=(nbr,), device_id_type=pl.DeviceIdType.MESH,
        )
    pl.semaphore_wait(barrier_sem, 2)

    def w_dma(e, slot):
        return pltpu.make_async_copy(
            w_hbm_ref.at[e], w_buf.at[slot], w_sems.at[slot]
        )

    w_dma(0, 0).start()
    xg[:, :] = jnp.dot(
        disp_ref[:, :], x_ref[:, :], preferred_element_type=jnp.float32
    ).astype(jnp.bfloat16)
    for e in range(EXP_PER_DEV):
        slot = e % 2
        w_dma(e, slot).wait()
        if e + 1 < EXP_PER_DEV:
            w_dma(e + 1, (e + 1) % 2).start()
        y_e = jnp.dot(
            xg[e * CAP_PAD:(e + 1) * CAP_PAD, :],
            w_buf[slot].astype(jnp.bfloat16),
            preferred_element_type=jnp.float32,
        ).astype(jnp.bfloat16)
        y_all[pl.ds(my_pos * ROWS + e * CAP_PAD, CAP_PAD), :] = y_e

    my_rows = pl.ds(my_pos * ROWS, ROWS)
    send_r = pltpu.make_async_remote_copy(
        src_ref=y_all.at[my_rows], dst_ref=y_all.at[my_rows],
        send_sem=send_sems.at[0], recv_sem=recv_sems.at[0],
        device_id=(right,), device_id_type=pl.DeviceIdType.MESH,
    )
    send_l = pltpu.make_async_remote_copy(
        src_ref=y_all.at[my_rows], dst_ref=y_all.at[my_rows],
        send_sem=send_sems.at[1], recv_sem=recv_sems.at[1],
        device_id=(left,), device_id_type=pl.DeviceIdType.MESH,
    )
    send_r.start()
    send_l.start()

    l_rows = pl.ds(left * ROWS, ROWS)
    recv_from_l = pltpu.make_async_remote_copy(
        src_ref=y_all.at[l_rows], dst_ref=y_all.at[l_rows],
        send_sem=send_sems.at[0], recv_sem=recv_sems.at[0],
        device_id=(left,), device_id_type=pl.DeviceIdType.MESH,
    )
    recv_from_l.wait_recv()

    l_top = pl.ds(left * ROWS, HALF)
    fwd_r = pltpu.make_async_remote_copy(
        src_ref=y_all.at[l_top], dst_ref=y_all.at[l_top],
        send_sem=send_sems.at[2], recv_sem=recv_sems.at[2],
        device_id=(right,), device_id_type=pl.DeviceIdType.MESH,
    )
    fwd_r.start()

    r_rows = pl.ds(right * ROWS, ROWS)
    recv_from_r = pltpu.make_async_remote_copy(
        src_ref=y_all.at[r_rows], dst_ref=y_all.at[r_rows],
        send_sem=send_sems.at[1], recv_sem=recv_sems.at[1],
        device_id=(right,), device_id_type=pl.DeviceIdType.MESH,
    )
    recv_from_r.wait_recv()

    r_bot = pl.ds(right * ROWS + HALF, HALF)
    fwd_l = pltpu.make_async_remote_copy(
        src_ref=y_all.at[r_bot], dst_ref=y_all.at[r_bot],
        send_sem=send_sems.at[3], recv_sem=recv_sems.at[3],
        device_id=(left,), device_id_type=pl.DeviceIdType.MESH,
    )
    fwd_l.start()

    d_top = pl.ds(diag * ROWS, HALF)
    d_bot = pl.ds(diag * ROWS + HALF, HALF)
    recv_d_top = pltpu.make_async_remote_copy(
        src_ref=y_all.at[d_top], dst_ref=y_all.at[d_top],
        send_sem=send_sems.at[2], recv_sem=recv_sems.at[2],
        device_id=(left,), device_id_type=pl.DeviceIdType.MESH,
    )
    recv_d_bot = pltpu.make_async_remote_copy(
        src_ref=y_all.at[d_bot], dst_ref=y_all.at[d_bot],
        send_sem=send_sems.at[3], recv_sem=recv_sems.at[3],
        device_id=(right,), device_id_type=pl.DeviceIdType.MESH,
    )
    recv_d_top.wait_recv()
    recv_d_bot.wait_recv()

    out_ref[:, :] = jnp.dot(
        unperm_ref[:, :], y_all[:, :], preferred_element_type=jnp.float32
    )

    send_r.wait_send()
    send_l.wait_send()
    fwd_r.wait_send()
    fwd_l.wait_send()


def _moe_pallas(x_bf, disp, unperm, expert_W):
    return pl.pallas_call(
        _moe_body,
        out_shape=jax.ShapeDtypeStruct((N_TOK, D_MODEL), jnp.float32),
        in_specs=[
            pl.BlockSpec(memory_space=pltpu.VMEM),
            pl.BlockSpec(memory_space=pltpu.VMEM),
            pl.BlockSpec(memory_space=pltpu.VMEM),
            pl.BlockSpec(memory_space=pl.ANY),
        ],
        out_specs=pl.BlockSpec(memory_space=pltpu.VMEM),
        scratch_shapes=[
            pltpu.VMEM((N_DEV * ROWS, D_MODEL), jnp.bfloat16),
            pltpu.VMEM((ROWS, D_MODEL), jnp.bfloat16),
            pltpu.VMEM((2, D_MODEL, D_MODEL), jnp.float32),
            pltpu.SemaphoreType.DMA((2,)),
            pltpu.SemaphoreType.DMA((4,)),
            pltpu.SemaphoreType.DMA((4,)),
        ],
        compiler_params=pltpu.CompilerParams(collective_id=0),
    )(x_bf, disp, unperm, expert_W)


def kernel(x, router_W, route_idx, expert_W):
    del router_W
    my_i = lax.axis_index("i")

    e = route_idx[:, 0]
    onehot = (e[:, None] == jnp.arange(N_EXP)[None, :]).astype(jnp.int32)
    rank = jnp.cumsum(onehot, axis=0) - onehot
    rank = jnp.sum(rank * onehot, axis=1)
    kept = rank < CAPACITY
    src_row = e * CAP_PAD + rank

    loc_rows = my_i * ROWS + jnp.arange(ROWS)
    disp = jnp.where(
        (src_row[None, :] == loc_rows[:, None]) & kept[None, :], 1.0, 0.0
    ).astype(jnp.bfloat16)
    unperm = jnp.where(
        (src_row[:, None] == jnp.arange(N_DEV * ROWS)[None, :])
        & kept[:, None],
        1.0,
        0.0,
    ).astype(jnp.bfloat16)

    return _moe_pallas(x.astype(jnp.bfloat16), disp, unperm, expert_W)


# device time: 56710 ns/iter; 1.6642x vs baseline; 1.4496x over previous
import jax
import jax.numpy as jnp
from jax import lax
from jax.experimental import pallas as pl
from jax.experimental.pallas import tpu as pltpu

N_DEV = 4
N_TOK = 2048
D_MODEL = 1024
N_EXP = 32
EXP_PER_DEV = N_EXP // N_DEV
CAPACITY = 51
CAP_PAD = 64
ROWS = EXP_PER_DEV * CAP_PAD
HALF = ROWS // 2

MINE, FROM_L, FROM_R, DIAG = 0, ROWS, 2 * ROWS, 3 * ROWS


def _moe_body(x_ref, disp_ref, unperm_ref, w_hbm_ref, out_ref,
              y_all, xb, xg, w_buf, w_sems,
              s_r, s_l, r_l, r_r, s_fwd, r_diag):
    my_pos = lax.axis_index("i")
    left = (my_pos - 1) % N_DEV
    right = (my_pos + 1) % N_DEV

    barrier_sem = pltpu.get_barrier_semaphore()
    for nbr in [left, right]:
        pl.semaphore_signal(
            barrier_sem, inc=1,
            device_id=(nbr,), device_id_type=pl.DeviceIdType.MESH,
        )
    pl.semaphore_wait(barrier_sem, 2)

    def w_dma(e, slot):
        return pltpu.make_async_copy(
            w_hbm_ref.at[e], w_buf.at[slot], w_sems.at[slot]
        )

    def chunk(base, e):
        return y_all.at[pl.ds(base + e * CAP_PAD, CAP_PAD)]

    def rdma(src, dst, ssem, rsem, dev):
        return pltpu.make_async_remote_copy(
            src_ref=src, dst_ref=dst, send_sem=ssem, recv_sem=rsem,
            device_id=(dev,), device_id_type=pl.DeviceIdType.MESH,
        )

    with jax.named_scope("dispatch"):
        w_dma(0, 0).start()
        xb[:, :] = x_ref[:, :].astype(jnp.bfloat16)
        xg[:, :] = jnp.dot(
            disp_ref[:, :], xb[:, :], preferred_element_type=jnp.float32
        ).astype(jnp.bfloat16)
    with jax.named_scope("experts"):
        for e in range(EXP_PER_DEV):
            slot = e % 2
            w_dma(e, slot).wait()
            if e + 1 < EXP_PER_DEV:
                w_dma(e + 1, (e + 1) % 2).start()
            y_e = jnp.dot(
                xg[e * CAP_PAD:(e + 1) * CAP_PAD, :],
                w_buf[slot].astype(jnp.bfloat16),
                preferred_element_type=jnp.float32,
            ).astype(jnp.bfloat16)
            y_all[MINE + e * CAP_PAD:MINE + (e + 1) * CAP_PAD, :] = y_e
            rdma(chunk(MINE, e), chunk(FROM_L, e),
                 s_r.at[e], r_l.at[e], right).start()
            rdma(chunk(MINE, e), chunk(FROM_R, e),
                 s_l.at[e], r_r.at[e], left).start()

    def unperm_acc(k, first=False):
        part = jnp.dot(
            unperm_ref[:, k * ROWS:(k + 1) * ROWS],
            y_all[k * ROWS:(k + 1) * ROWS, :],
            preferred_element_type=jnp.float32,
        )
        if first:
            out_ref[:, :] = part
        else:
            out_ref[:, :] += part

    with jax.named_scope("unperm0"):
        unperm_acc(0, first=True)

    with jax.named_scope("p1_wait_l"):
        for e in range(EXP_PER_DEV):
            rdma(chunk(FROM_L, e), chunk(FROM_L, e),
                 s_r.at[e], r_l.at[e], left).wait_recv()
    fwd_r = rdma(y_all.at[pl.ds(FROM_L, HALF)],
                 y_all.at[pl.ds(DIAG, HALF)],
                 s_fwd.at[0], r_diag.at[0], right)
    fwd_r.start()
    with jax.named_scope("unperm1"):
        unperm_acc(1)

    with jax.named_scope("p1_wait_r"):
        for e in range(EXP_PER_DEV):
            rdma(chunk(FROM_R, e), chunk(FROM_R, e),
                 s_l.at[e], r_r.at[e], right).wait_recv()
    fwd_l = rdma(y_all.at[pl.ds(FROM_R + HALF, HALF)],
                 y_all.at[pl.ds(DIAG + HALF, HALF)],
                 s_fwd.at[1], r_diag.at[1], left)
    fwd_l.start()
    with jax.named_scope("unperm2"):
        unperm_acc(2)

    with jax.named_scope("p2_wait_diag"):
        rdma(y_all.at[pl.ds(DIAG, HALF)], y_all.at[pl.ds(DIAG, HALF)],
             s_fwd.at[0], r_diag.at[0], left).wait_recv()
        rdma(y_all.at[pl.ds(DIAG + HALF, HALF)],
             y_all.at[pl.ds(DIAG + HALF, HALF)],
             s_fwd.at[1], r_diag.at[1], right).wait_recv()
    with jax.named_scope("unperm3"):
        unperm_acc(3)

    with jax.named_scope("drain"):
        for e in range(EXP_PER_DEV):
            rdma(chunk(MINE, e), chunk(FROM_L, e),
                 s_r.at[e], r_l.at[e], right).wait_send()
            rdma(chunk(MINE, e), chunk(FROM_R, e),
                 s_l.at[e], r_r.at[e], left).wait_send()
        fwd_r.wait_send()
        fwd_l.wait_send()


def _moe_pallas(x, disp, unperm, expert_W):
    return pl.pallas_call(
        _moe_body,
        out_shape=jax.ShapeDtypeStruct((N_TOK, D_MODEL), jnp.float32),
        in_specs=[
            pl.BlockSpec(memory_space=pltpu.VMEM),
            pl.BlockSpec(memory_space=pltpu.VMEM),
            pl.BlockSpec(memory_space=pltpu.VMEM),
            pl.BlockSpec(memory_space=pl.ANY),
        ],
        out_specs=pl.BlockSpec(memory_space=pltpu.VMEM),
        scratch_shapes=[
            pltpu.VMEM((N_DEV * ROWS, D_MODEL), jnp.bfloat16),
            pltpu.VMEM((N_TOK, D_MODEL), jnp.bfloat16),
            pltpu.VMEM((ROWS, D_MODEL), jnp.bfloat16),
            pltpu.VMEM((2, D_MODEL, D_MODEL), jnp.float32),
            pltpu.SemaphoreType.DMA((2,)),
            pltpu.SemaphoreType.DMA((EXP_PER_DEV,)),
            pltpu.SemaphoreType.DMA((EXP_PER_DEV,)),
            pltpu.SemaphoreType.DMA((EXP_PER_DEV,)),
            pltpu.SemaphoreType.DMA((EXP_PER_DEV,)),
            pltpu.SemaphoreType.DMA((2,)),
            pltpu.SemaphoreType.DMA((2,)),
        ],
        compiler_params=pltpu.CompilerParams(collective_id=0),
    )(x, disp, unperm, expert_W)


def kernel(x, router_W, route_idx, expert_W):
    del router_W
    my_i = lax.axis_index("i")

    e = route_idx[:, 0]
    B, T = 64, 32
    oh = (e[:, None] == jnp.arange(N_EXP)[None, :]).astype(jnp.float32)
    oh3 = oh.reshape(B, T, N_EXP)
    within = jnp.einsum(
        "ts,bse->bte", jnp.tril(jnp.ones((T, T), jnp.float32), -1), oh3
    )
    prefix = jnp.dot(
        jnp.tril(jnp.ones((B, B), jnp.float32), -1), oh3.sum(axis=1)
    )
    rank3 = within + prefix[:, None, :]
    rank = jnp.sum(rank3 * oh3, axis=2).reshape(N_TOK).astype(jnp.int32)
    kept = rank < CAPACITY
    src_row = e * CAP_PAD + rank

    loc_rows = my_i * ROWS + jnp.arange(ROWS)
    disp = jnp.where(
        (src_row[None, :] == loc_rows[:, None]) & kept[None, :], 1.0, 0.0
    ).astype(jnp.bfloat16)

    dev_order = jnp.stack(
        [my_i, (my_i - 1) % N_DEV, (my_i + 1) % N_DEV, (my_i + 2) % N_DEV]
    )
    col_g = (dev_order[:, None] * ROWS + jnp.arange(ROWS)[None, :]).reshape(-1)
    unperm = jnp.where(
        (src_row[:, None] == col_g[None, :]) & kept[:, None], 1.0, 0.0
    ).astype(jnp.bfloat16)

    return _moe_pallas(x, disp, unperm, expert_W)
